# Initial kernel scaffold; baseline (speedup 1.0000x reference)
import jax
import jax.numpy as jnp
from jax.experimental import pallas as pl


def kernel(x, edge_index, edge_attr, enc_n_W0, enc_n_b0, enc_n_W1, enc_n_b1, enc_e_W0, enc_e_b0, enc_e_W1, enc_e_b1, t, mlp_W0, mlp_b0, mlp_g, mlp_bn, mlp_W1, mlp_b1, ln_g, ln_b, dec_n_W0, dec_n_b0, dec_n_W1, dec_n_b1, dec_e_W0, dec_e_b0, dec_e_W1, dec_e_b1):
    raise NotImplementedError("write your pallas kernel here")


# speedup vs baseline: 1.0229x; 1.0229x over previous
import jax
import jax.numpy as jnp
from jax.experimental import pallas as pl

N = 10000


def _mlp2(x, W0, b0, W1, b1):
    return jnp.maximum(x @ W0 + b0, 0.0) @ W1 + b1


def _layer_norm(x, g, b):
    m = jnp.mean(x, axis=-1, keepdims=True)
    v = jnp.mean((x - m) ** 2, axis=-1, keepdims=True)
    return (x - m) / jnp.sqrt(v + 1e-5) * g + b


def _seg_softmax(x, seg, n):
    xm = jax.ops.segment_max(x, seg, num_segments=n)
    xm = jnp.where(jnp.isfinite(xm), xm, 0.0)
    ex = jnp.exp(x - xm[seg])
    den = jax.ops.segment_sum(ex, seg, num_segments=n)
    return ex / (den[seg] + 1e-16)


def _gen_conv(h, src, dst, e, t, W0, b0, g, bn, W1, b1):
    m = jnp.maximum(h[src] + e, 0.0) + 1e-7
    alpha = _seg_softmax(m * t, dst, N)
    agg = jax.ops.segment_sum(m * alpha, dst, num_segments=N)
    out = agg + h
    z = out @ W0 + b0
    z = jnp.maximum(_layer_norm(z, g, bn), 0.0)
    return z @ W1 + b1


def _add_kernel(a_ref, b_ref, o_ref):
    o_ref[...] = a_ref[...] + b_ref[...]


def _padd(a, b):
    return pl.pallas_call(
        _add_kernel,
        out_shape=jax.ShapeDtypeStruct(a.shape, a.dtype),
    )(a, b)


def kernel(x, edge_index, edge_attr, enc_n_W0, enc_n_b0, enc_n_W1, enc_n_b1, enc_e_W0, enc_e_b0, enc_e_W1, enc_e_b1, t, mlp_W0, mlp_b0, mlp_g, mlp_bn, mlp_W1, mlp_b1, ln_g, ln_b, dec_n_W0, dec_n_b0, dec_n_W1, dec_n_b1, dec_e_W0, dec_e_b0, dec_e_W1, dec_e_b1):
    src, dst = edge_index[0], edge_index[1]
    h = _mlp2(x, enc_n_W0, enc_n_b0, enc_n_W1, enc_n_b1)
    e = _mlp2(edge_attr, enc_e_W0, enc_e_b0, enc_e_W1, enc_e_b1)
    h = _gen_conv(h, src, dst, e, t[0], mlp_W0[0], mlp_b0[0], mlp_g[0], mlp_bn[0], mlp_W1[0], mlp_b1[0])
    for i in range(1, 3):
        r = jnp.maximum(_layer_norm(h, ln_g[i], ln_b[i]), 0.0)
        r = _gen_conv(r, src, dst, e, t[i], mlp_W0[i], mlp_b0[i], mlp_g[i], mlp_bn[i], mlp_W1[i], mlp_b1[i])
        h = _padd(h, r)
    h = jnp.maximum(_layer_norm(h, ln_g[0], ln_b[0]), 0.0)
    ecat = jnp.concatenate([h[src], h[dst]], axis=-1)
    x_out = _mlp2(h, dec_n_W0, dec_n_b0, dec_n_W1, dec_n_b1)
    e_out = _mlp2(ecat, dec_e_W0, dec_e_b0, dec_e_W1, dec_e_b1)
    return (x_out, e_out)


# speedup vs baseline: 4.0985x; 4.0069x over previous
import functools

import jax
import jax.numpy as jnp
from jax import lax
from jax.experimental import pallas as pl
from jax.experimental.pallas import tpu as pltpu
from jax.experimental.pallas import tpu_sc as plsc

_N = 10000
_E = 320000
_H = 64
_NC = 2
_NS = 16
_NW = _NC * _NS
_EPW = _E // _NW
_CH = 128
_NFULL = _EPW // _CH
_TAIL = _EPW - _NFULL * _CH
_NPAD = 10240


def _mlp_enc_call(x, W0, b0, W1, b1, bm, pad_out):
    M, K = x.shape
    K2 = W1.shape[0]
    Dout = W1.shape[1]
    Wout = 2 * Dout if pad_out else Dout
    grid = (M // bm,)

    def kfn(x_ref, w0_ref, b0_ref, w1_ref, b1_ref, o_ref, mx_ref):
        z = jnp.dot(x_ref[...], w0_ref[...], preferred_element_type=jnp.float32)
        z = jnp.maximum(z + b0_ref[...], 0.0)
        h = jnp.dot(z, w1_ref[...], preferred_element_type=jnp.float32) + b1_ref[...]
        if pad_out:
            o_ref[...] = jnp.concatenate([h, jnp.zeros_like(h)], axis=-1)
        else:
            o_ref[...] = h
        cur = jnp.max(h, axis=0, keepdims=True)

        @pl.when(pl.program_id(0) == 0)
        def _():
            mx_ref[...] = cur

        @pl.when(pl.program_id(0) != 0)
        def _():
            mx_ref[...] = jnp.maximum(mx_ref[...], cur)

    return pl.pallas_call(
        kfn,
        grid=grid,
        in_specs=[
            pl.BlockSpec((bm, K), lambda i: (i, 0)),
            pl.BlockSpec((K, K2), lambda i: (0, 0)),
            pl.BlockSpec((1, K2), lambda i: (0, 0)),
            pl.BlockSpec((K2, Dout), lambda i: (0, 0)),
            pl.BlockSpec((1, Dout), lambda i: (0, 0)),
        ],
        out_specs=[
            pl.BlockSpec((bm, Wout), lambda i: (i, 0)),
            pl.BlockSpec((1, Dout), lambda i: (0, 0)),
        ],
        out_shape=[
            jax.ShapeDtypeStruct((M, Wout), jnp.float32),
            jax.ShapeDtypeStruct((1, Dout), jnp.float32),
        ],
    )(x, W0, b0.reshape(1, -1), W1, b1.reshape(1, -1))


def _ln(z, g, b):
    mu = jnp.mean(z, axis=-1, keepdims=True)
    var = jnp.mean((z - mu) ** 2, axis=-1, keepdims=True)
    return (z - mu) * jax.lax.rsqrt(var + 1e-5) * g + b


def _layer_post_call(mode, acc0, acc1, hin, hprev,
                     W0, b0, g, bn, W1, b1, g2, b2, dW0, db0, dW1, db1):
    bm = 1000
    grid = (_N // bm,)

    def kfn(*refs):
        if mode == 2:
            (a0, a1, hi, hp, w0, b0r, gr, bnr, w1, b1r, g2r, b2r,
             dw0, db0r, dw1, db1r, o0, o1) = refs
        elif mode == 1:
            (a0, a1, hi, hp, w0, b0r, gr, bnr, w1, b1r, g2r, b2r,
             o0, o1, o2) = refs
        else:
            (a0, a1, hi, w0, b0r, gr, bnr, w1, b1r, g2r, b2r,
             o0, o1, o2) = refs
            hp = None
        av0 = a0[...]
        av1 = a1[...]
        num = av0[:, :_H] + av1[:, :_H]
        den = av0[:, _H:] + av1[:, _H:]
        ok = den > 0.0
        agg = jnp.where(ok, num / jnp.where(ok, den, 1.0), 0.0)
        out = agg + hi[...][:, :_H]
        z = jnp.dot(out, w0[...], preferred_element_type=jnp.float32) + b0r[...]
        z = jnp.maximum(_ln(z, gr[...], bnr[...]), 0.0)
        conv = jnp.dot(z, w1[...], preferred_element_type=jnp.float32) + b1r[...]
        if mode == 0:
            h_new = conv
        else:
            h_new = hp[...][:, :_H] + conv
        post = jnp.maximum(_ln(h_new, g2r[...], b2r[...]), 0.0)
        zpad = jnp.zeros_like(post)
        if mode == 2:
            o0[...] = jnp.concatenate([post, zpad], axis=-1)
            zz = jnp.dot(post, dw0[...], preferred_element_type=jnp.float32)
            zz = jnp.maximum(zz + db0r[...], 0.0)
            o1[...] = jnp.dot(zz, dw1[...], preferred_element_type=jnp.float32) + db1r[...]
        else:
            o0[...] = jnp.concatenate([h_new, zpad], axis=-1)
            o1[...] = jnp.concatenate([post, zpad], axis=-1)
            cur = jnp.max(post, axis=0, keepdims=True)

            @pl.when(pl.program_id(0) == 0)
            def _():
                o2[...] = cur

            @pl.when(pl.program_id(0) != 0)
            def _():
                o2[...] = jnp.maximum(o2[...], cur)

    wide_spec = pl.BlockSpec((bm, 2 * _H), lambda i: (i, 0))
    full = lambda r, c: pl.BlockSpec((r, c), lambda i: (0, 0))
    in_specs = [wide_spec, wide_spec, wide_spec]
    args = [acc0, acc1, hin]
    if mode != 0:
        in_specs.append(wide_spec)
        args.append(hprev)
    in_specs += [full(_H, 2 * _H), full(1, 2 * _H), full(1, 2 * _H),
                 full(1, 2 * _H), full(2 * _H, _H), full(1, _H),
                 full(1, _H), full(1, _H)]
    args += [W0, b0.reshape(1, -1), g.reshape(1, -1), bn.reshape(1, -1),
             W1, b1.reshape(1, -1), g2.reshape(1, -1), b2.reshape(1, -1)]
    if mode == 2:
        in_specs += [full(_H, 128), full(1, 128), full(128, 2), full(1, 2)]
        args += [dW0, db0.reshape(1, -1), dW1, db1.reshape(1, -1)]
        out_specs = [wide_spec, pl.BlockSpec((bm, 2), lambda i: (i, 0))]
        out_shape = [jax.ShapeDtypeStruct((_N, 2 * _H), jnp.float32),
                     jax.ShapeDtypeStruct((_N, 2), jnp.float32)]
    else:
        out_specs = [wide_spec, wide_spec, pl.BlockSpec((1, _H), lambda i: (0, 0))]
        out_shape = [jax.ShapeDtypeStruct((_N, 2 * _H), jnp.float32),
                     jax.ShapeDtypeStruct((_N, 2 * _H), jnp.float32),
                     jax.ShapeDtypeStruct((1, _H), jnp.float32)]

    return pl.pallas_call(
        kfn, grid=grid, in_specs=in_specs, out_specs=out_specs,
        out_shape=out_shape,
    )(*args)


def _dec_edge_call(Gcat, W0, b0, W1, b1):
    be = 2000
    grid = (_E // be,)

    def kfn(gc, w0, b0r, w1, b1r, o_ref):
        z = jnp.dot(gc[...], w0[...], preferred_element_type=jnp.float32)
        z = jnp.maximum(z + b0r[...], 0.0)
        o_ref[...] = jnp.dot(z, w1[...], preferred_element_type=jnp.float32) + b1r[...]

    full = lambda r, c: pl.BlockSpec((r, c), lambda i: (0, 0))
    return pl.pallas_call(
        kfn, grid=grid,
        in_specs=[
            pl.BlockSpec((be, 2 * _H), lambda i: (i, 0)),
            full(2 * _H, 256), full(1, 256), full(256, 2), full(1, 2),
        ],
        out_specs=pl.BlockSpec((be, 2), lambda i: (i, 0)),
        out_shape=jax.ShapeDtypeStruct((_E, 2), jnp.float32),
    )(Gcat, W0, b0.reshape(1, -1), W1, b1.reshape(1, -1))


def _sc_mesh():
    return plsc.VectorSubcoreMesh(core_axis_name="c", subcore_axis_name="s")


def _sc_agg_call(h, e, src, dst, c64, t16, zpad):
    @functools.partial(
        pl.kernel,
        mesh=_sc_mesh(),
        out_type=jax.ShapeDtypeStruct((_NC, _NPAD, 2 * _H), jnp.float32),
        scratch_types=[
            pltpu.VMEM((_CH,), jnp.int32),
            pltpu.VMEM((_CH,), jnp.int32),
            pltpu.VMEM((_CH, 2 * _H), jnp.float32),
            pltpu.VMEM((_CH, _H), jnp.float32),
            pltpu.VMEM((_TAIL,), jnp.int32),
            pltpu.VMEM((_TAIL,), jnp.int32),
            pltpu.VMEM((_TAIL, 2 * _H), jnp.float32),
            pltpu.VMEM((_TAIL, _H), jnp.float32),
            pltpu.VMEM((_H,), jnp.float32),
            pltpu.VMEM((16,), jnp.float32),
            pltpu.VMEM_SHARED((_NPAD, 2 * _H), jnp.float32),
            pltpu.SemaphoreType.DMA,
        ],
    )
    def k(h_hbm, e_hbm, src_hbm, dst_hbm, c_hbm, t_hbm, z_hbm, acc_out,
          src_v, dst_v, hrow, erow, src_tv, dst_tv, hrow_t, erow_t,
          c_v, t_v, acc_sh, sem):
        cid = lax.axis_index("c")
        sid = lax.axis_index("s")
        wid = sid * _NC + cid
        rows = _NPAD // _NS
        pltpu.sync_copy(z_hbm.at[pl.ds(sid * rows, rows)],
                        acc_sh.at[pl.ds(sid * rows, rows)])
        pltpu.sync_copy(c_hbm, c_v)
        pltpu.sync_copy(t_hbm, t_v)
        plsc.subcore_barrier()
        base0 = wid * _EPW

        def compute(hr, er, nrows):
            def body(j, carry):
                tv = t_v[...]
                for q in range(4):
                    cq = c_v[pl.ds(q * 16, 16)]
                    hv = hr[j, pl.ds(q * 16, 16)]
                    ev = er[j, pl.ds(q * 16, 16)]
                    m = jnp.maximum(hv + ev, 0.0) + 1e-7
                    em = jnp.exp(m * tv - cq)
                    hr[j, pl.ds(q * 16, 16)] = m * em
                    hr[j, pl.ds(_H + q * 16, 16)] = em
                return carry
            lax.fori_loop(0, nrows, body, 0)

        def chunk(base, sv, dv, hr, er, nrows):
            pltpu.sync_copy(src_hbm.at[pl.ds(base, nrows)], sv)
            pltpu.sync_copy(dst_hbm.at[pl.ds(base, nrows)], dv)
            pltpu.async_copy(h_hbm.at[sv], hr, sem).wait()
            pltpu.sync_copy(e_hbm.at[pl.ds(base, nrows)], er)
            compute(hr, er, nrows)
            pltpu.sync_copy(hr, acc_sh.at[dv], add=True)

        def loop_body(i, carry):
            chunk(base0 + i * _CH, src_v, dst_v, hrow, erow, _CH)
            return carry
        lax.fori_loop(0, _NFULL, loop_body, 0)
        chunk(base0 + _NFULL * _CH, src_tv, dst_tv, hrow_t, erow_t, _TAIL)

        plsc.subcore_barrier()
        pltpu.sync_copy(acc_sh.at[pl.ds(sid * rows, rows)],
                        acc_out.at[cid, pl.ds(sid * rows, rows)])

    return k(h, e, src, dst, c64, t16, zpad)


def _sc_ecat_call(h, src, dst):
    @functools.partial(
        pl.kernel,
        mesh=_sc_mesh(),
        out_type=jax.ShapeDtypeStruct((_E, 2 * _H), jnp.float32),
        scratch_types=[
            pltpu.VMEM((_CH,), jnp.int32),
            pltpu.VMEM((_CH, 2 * _H), jnp.float32),
            pltpu.VMEM((_CH, 2 * _H), jnp.float32),
            pltpu.VMEM((_TAIL,), jnp.int32),
            pltpu.VMEM((_TAIL, 2 * _H), jnp.float32),
            pltpu.VMEM((_TAIL, 2 * _H), jnp.float32),
            pltpu.SemaphoreType.DMA,
            pltpu.SemaphoreType.DMA,
        ],
    )
    def k(h_hbm, src_hbm, dst_hbm, gcat_out,
          idx_v, srow, drow, idx_tv, srow_t, drow_t, sem, sem2):
        cid = lax.axis_index("c")
        sid = lax.axis_index("s")
        wid = sid * _NC + cid
        base0 = wid * _EPW

        def chunk(base, iv, sr, dr, nrows):
            pltpu.sync_copy(src_hbm.at[pl.ds(base, nrows)], iv)
            pltpu.async_copy(h_hbm.at[iv], sr, sem).wait()
            pltpu.sync_copy(dst_hbm.at[pl.ds(base, nrows)], iv)
            pltpu.async_copy(h_hbm.at[iv], dr, sem2).wait()

            def body(j, carry):
                for q in range(4):
                    sr[j, pl.ds(_H + q * 16, 16)] = dr[j, pl.ds(q * 16, 16)]
                return carry
            lax.fori_loop(0, nrows, body, 0)
            pltpu.sync_copy(sr, gcat_out.at[pl.ds(base, nrows)])

        def loop_body(i, carry):
            chunk(base0 + i * _CH, idx_v, srow, drow, _CH)
            return carry
        lax.fori_loop(0, _NFULL, loop_body, 0)
        chunk(base0 + _NFULL * _CH, idx_tv, srow_t, drow_t, _TAIL)

    return k(h, src, dst)


def kernel(x, edge_index, edge_attr, enc_n_W0, enc_n_b0, enc_n_W1, enc_n_b1,
           enc_e_W0, enc_e_b0, enc_e_W1, enc_e_b1, t, mlp_W0, mlp_b0, mlp_g,
           mlp_bn, mlp_W1, mlp_b1, ln_g, ln_b, dec_n_W0, dec_n_b0, dec_n_W1,
           dec_n_b1, dec_e_W0, dec_e_b0, dec_e_W1, dec_e_b1):
    src, dst = edge_index[0], edge_index[1]
    zpad = jnp.zeros((_NPAD, 2 * _H), jnp.float32)

    h, hmax = _mlp_enc_call(x, enc_n_W0, enc_n_b0, enc_n_W1, enc_n_b1,
                            bm=1000, pad_out=True)
    e, emax = _mlp_enc_call(edge_attr, enc_e_W0, enc_e_b0, enc_e_W1,
                            enc_e_b1, bm=2000, pad_out=False)

    hin = h
    hprev = None
    cmax = hmax
    for i in range(3):
        c = jnp.maximum(t[i] * (jnp.maximum(cmax[0] + emax[0], 0.0) + 1e-7),
                        0.0)
        t16 = jnp.full((16,), t[i], jnp.float32)
        acc = _sc_agg_call(hin, e, src, dst, c, t16, zpad)
        mode = 0 if i == 0 else (1 if i == 1 else 2)
        if mode == 2:
            hfin, x_out = _layer_post_call(
                2, acc[0], acc[1], hin, hprev,
                mlp_W0[i], mlp_b0[i], mlp_g[i], mlp_bn[i], mlp_W1[i],
                mlp_b1[i], ln_g[0], ln_b[0],
                dec_n_W0, dec_n_b0, dec_n_W1, dec_n_b1)
        else:
            h_new, r_next, rmax = _layer_post_call(
                mode, acc[0], acc[1], hin, hprev,
                mlp_W0[i], mlp_b0[i], mlp_g[i], mlp_bn[i], mlp_W1[i],
                mlp_b1[i], ln_g[i + 1], ln_b[i + 1],
                None, None, None, None)
            hprev = h_new
            hin = r_next
            cmax = rmax

    Gcat = _sc_ecat_call(hfin, src, dst)
    e_out = _dec_edge_call(Gcat, dec_e_W0, dec_e_b0, dec_e_W1, dec_e_b1)
    return (x_out, e_out)


# speedup vs baseline: 5.2925x; 1.2913x over previous
import functools

import jax
import jax.numpy as jnp
from jax import lax
from jax.experimental import pallas as pl
from jax.experimental.pallas import tpu as pltpu
from jax.experimental.pallas import tpu_sc as plsc

_N = 10000
_E = 320000
_H = 64
_NC = 2
_NS = 16
_NW = _NC * _NS
_CH = 80
_NCHUNK = _E // _CH
_UCH = 126
_IPAD = _NW * _UCH
_DUMP = 10239
_NPAD = 10240


def _mlp_enc_call(x, W0, b0, W1, b1, bm, pad_out):
    M, K = x.shape
    K2 = W1.shape[0]
    Dout = W1.shape[1]
    Wout = 2 * Dout if pad_out else Dout
    grid = (M // bm,)

    def kfn(x_ref, w0_ref, b0_ref, w1_ref, b1_ref, o_ref, mx_ref):
        z = jnp.dot(x_ref[...], w0_ref[...], preferred_element_type=jnp.float32)
        z = jnp.maximum(z + b0_ref[...], 0.0)
        h = jnp.dot(z, w1_ref[...], preferred_element_type=jnp.float32) + b1_ref[...]
        if pad_out:
            o_ref[...] = jnp.concatenate([h, jnp.zeros_like(h)], axis=-1)
        else:
            o_ref[...] = h
        cur = jnp.max(h, axis=0, keepdims=True)

        @pl.when(pl.program_id(0) == 0)
        def _():
            mx_ref[...] = cur

        @pl.when(pl.program_id(0) != 0)
        def _():
            mx_ref[...] = jnp.maximum(mx_ref[...], cur)

    return pl.pallas_call(
        kfn,
        grid=grid,
        in_specs=[
            pl.BlockSpec((bm, K), lambda i: (i, 0)),
            pl.BlockSpec((K, K2), lambda i: (0, 0)),
            pl.BlockSpec((1, K2), lambda i: (0, 0)),
            pl.BlockSpec((K2, Dout), lambda i: (0, 0)),
            pl.BlockSpec((1, Dout), lambda i: (0, 0)),
        ],
        out_specs=[
            pl.BlockSpec((bm, Wout), lambda i: (i, 0)),
            pl.BlockSpec((1, Dout), lambda i: (0, 0)),
        ],
        out_shape=[
            jax.ShapeDtypeStruct((M, Wout), jnp.float32),
            jax.ShapeDtypeStruct((1, Dout), jnp.float32),
        ],
    )(x, W0, b0.reshape(1, -1), W1, b1.reshape(1, -1))


def _ln(z, g, b):
    mu = jnp.mean(z, axis=-1, keepdims=True)
    var = jnp.mean((z - mu) ** 2, axis=-1, keepdims=True)
    return (z - mu) * jax.lax.rsqrt(var + 1e-5) * g + b


def _layer_post_call(acc0, acc1, hin, hprev, W0, b0, g, bn, W1, b1, g2, b2):
    bm = 1000
    grid = (_N // bm,)

    def kfn(a0, a1, hi, hp, w0, b0r, gr, bnr, w1, b1r, g2r, b2r,
            o0, o1, o2):
        av0 = a0[...]
        av1 = a1[...]
        num = av0[:, :_H] + av1[:, :_H]
        den = av0[:, _H:] + av1[:, _H:]
        ok = den > 0.0
        agg = jnp.where(ok, num / jnp.where(ok, den, 1.0), 0.0)
        out = agg + hi[...][:, :_H]
        z = jnp.dot(out, w0[...], preferred_element_type=jnp.float32) + b0r[...]
        z = jnp.maximum(_ln(z, gr[...], bnr[...]), 0.0)
        conv = jnp.dot(z, w1[...], preferred_element_type=jnp.float32) + b1r[...]
        h_new = hp[...][:, :_H] + conv
        post = jnp.maximum(_ln(h_new, g2r[...], b2r[...]), 0.0)
        zpad = jnp.zeros_like(post)
        o0[...] = jnp.concatenate([h_new, zpad], axis=-1)
        o1[...] = jnp.concatenate([post, zpad], axis=-1)
        cur = jnp.max(post, axis=0, keepdims=True)

        @pl.when(pl.program_id(0) == 0)
        def _():
            o2[...] = cur

        @pl.when(pl.program_id(0) != 0)
        def _():
            o2[...] = jnp.maximum(o2[...], cur)

    wide_spec = pl.BlockSpec((bm, 2 * _H), lambda i: (i, 0))
    full = lambda r, c: pl.BlockSpec((r, c), lambda i: (0, 0))
    in_specs = [wide_spec, wide_spec, wide_spec, wide_spec,
                full(_H, 2 * _H), full(1, 2 * _H), full(1, 2 * _H),
                full(1, 2 * _H), full(2 * _H, _H), full(1, _H),
                full(1, _H), full(1, _H)]
    args = [acc0, acc1, hin, hprev,
            W0, b0.reshape(1, -1), g.reshape(1, -1), bn.reshape(1, -1),
            W1, b1.reshape(1, -1), g2.reshape(1, -1), b2.reshape(1, -1)]
    out_specs = [wide_spec, wide_spec, pl.BlockSpec((1, _H), lambda i: (0, 0))]
    out_shape = [jax.ShapeDtypeStruct((_N, 2 * _H), jnp.float32),
                 jax.ShapeDtypeStruct((_N, 2 * _H), jnp.float32),
                 jax.ShapeDtypeStruct((1, _H), jnp.float32)]

    return pl.pallas_call(
        kfn, grid=grid, in_specs=in_specs, out_specs=out_specs,
        out_shape=out_shape,
    )(*args)


def _dec_node_call(hfin, dW0, db0, dW1, db1):
    bm = 1000
    grid = (_N // bm,)

    def kfn(hi, dw0, db0r, dw1, db1r, o_ref):
        hv = hi[...][:, :_H]
        zz = jnp.dot(hv, dw0[...], preferred_element_type=jnp.float32)
        zz = jnp.maximum(zz + db0r[...], 0.0)
        o_ref[...] = jnp.dot(zz, dw1[...], preferred_element_type=jnp.float32) + db1r[...]

    full = lambda r, c: pl.BlockSpec((r, c), lambda i: (0, 0))
    return pl.pallas_call(
        kfn, grid=grid,
        in_specs=[pl.BlockSpec((bm, 2 * _H), lambda i: (i, 0)),
                  full(_H, 128), full(1, 128), full(128, 2), full(1, 2)],
        out_specs=pl.BlockSpec((bm, 2), lambda i: (i, 0)),
        out_shape=jax.ShapeDtypeStruct((_N, 2), jnp.float32),
    )(hfin, dW0, db0.reshape(1, -1), dW1, db1.reshape(1, -1))


def _dec_edge_call(Gcat, W0, b0, W1, b1):
    be = 2000
    grid = (_E // be,)

    def kfn(gc, w0, b0r, w1, b1r, o_ref):
        z = jnp.dot(gc[...], w0[...], preferred_element_type=jnp.float32)
        z = jnp.maximum(z + b0r[...], 0.0)
        o_ref[...] = jnp.dot(z, w1[...], preferred_element_type=jnp.float32) + b1r[...]

    full = lambda r, c: pl.BlockSpec((r, c), lambda i: (0, 0))
    return pl.pallas_call(
        kfn, grid=grid,
        in_specs=[
            pl.BlockSpec((be, 2 * _H), lambda i: (i, 0)),
            full(2 * _H, 256), full(1, 256), full(256, 2), full(1, 2),
        ],
        out_specs=pl.BlockSpec((be, 2), lambda i: (i, 0)),
        out_shape=jax.ShapeDtypeStruct((_E, 2), jnp.float32),
    )(Gcat, W0, b0.reshape(1, -1), W1, b1.reshape(1, -1))


def _sc_mesh():
    return plsc.VectorSubcoreMesh(core_axis_name="c", subcore_axis_name="s")


def _sc_agg_call(h, e, src2, dst2, c64, t16, zpad):
    @functools.partial(
        pl.kernel,
        mesh=_sc_mesh(),
        out_type=jax.ShapeDtypeStruct((_NC, _NPAD, 2 * _H), jnp.float32),
        scratch_types=[
            pltpu.VMEM((2, 1, _CH), jnp.int32),
            pltpu.VMEM((2, 1, _CH), jnp.int32),
            pltpu.VMEM((_CH, 2 * _H), jnp.float32),
            pltpu.VMEM((_CH, 2 * _H), jnp.float32),
            pltpu.VMEM((_CH, _H), jnp.float32),
            pltpu.VMEM((_CH, _H), jnp.float32),
            pltpu.VMEM((_H,), jnp.float32),
            pltpu.VMEM((16,), jnp.float32),
            pltpu.VMEM_SHARED((_NPAD, 2 * _H), jnp.float32),
            pltpu.SemaphoreType.DMA,
            pltpu.SemaphoreType.DMA,
            pltpu.SemaphoreType.DMA,
            pltpu.SemaphoreType.DMA,
        ],
    )
    def k(h_hbm, e_hbm, src_hbm, dst_hbm, c_hbm, t_hbm, z_hbm, acc_out,
          srcp, dstp, hrow0, hrow1, erow0, erow1,
          c_v, t_v, acc_sh, gsem0, gsem1, esem0, esem1):
        cid = lax.axis_index("c")
        sid = lax.axis_index("s")
        wid = sid * _NC + cid
        rows = _NPAD // _NS
        for zi in range(rows // _CH):
            pltpu.sync_copy(
                z_hbm, acc_sh.at[pl.ds(sid * rows + zi * _CH, _CH)])
        pltpu.sync_copy(c_hbm, c_v)
        pltpu.sync_copy(t_hbm, t_v)
        start = _UCH * wid
        plsc.subcore_barrier()

        def compute(hr, er):
            def body(i, carry):
                c0, c1, c2, c3, tv = carry
                cs = (c0, c1, c2, c3)
                for q in range(4):
                    hv = hr[i, pl.ds(q * 16, 16)]
                    ev = er[i, pl.ds(q * 16, 16)]
                    m = jnp.maximum(hv + ev, 0.0) + 1e-7
                    em = jnp.exp(m * tv - cs[q])
                    hr[i, pl.ds(q * 16, 16)] = m * em
                    hr[i, pl.ds(_H + q * 16, 16)] = em
                return carry
            carry0 = (c_v[pl.ds(0, 16)], c_v[pl.ds(16, 16)],
                      c_v[pl.ds(32, 16)], c_v[pl.ds(48, 16)], t_v[...])
            lax.fori_loop(0, _CH, body, carry0)

        def pair(p, carry):
            j0 = 2 * p
            g0 = start + j0
            g1 = g0 + 1
            ge0 = jnp.where(g0 < _NCHUNK, g0, 0)
            ge1 = jnp.where(g1 < _NCHUNK, g1, 0)
            pltpu.sync_copy(src_hbm.at[pl.ds(g0, 2)], srcp)
            pltpu.sync_copy(dst_hbm.at[pl.ds(g0, 2)], dstp)
            cp0 = pltpu.async_copy(h_hbm.at[srcp.at[0, 0]], hrow0, gsem0)
            cp1 = pltpu.async_copy(e_hbm.at[pl.ds(ge0 * _CH, _CH)], erow0,
                                   esem0)
            cp2 = pltpu.async_copy(h_hbm.at[srcp.at[1, 0]], hrow1, gsem1)
            cp3 = pltpu.async_copy(e_hbm.at[pl.ds(ge1 * _CH, _CH)], erow1,
                                   esem1)
            cp0.wait()
            cp1.wait()
            compute(hrow0, erow0)
            pltpu.sync_copy(hrow0, acc_sh.at[dstp.at[0, 0]], add=True)
            cp2.wait()
            cp3.wait()
            compute(hrow1, erow1)
            pltpu.sync_copy(hrow1, acc_sh.at[dstp.at[1, 0]], add=True)
            return carry
        lax.fori_loop(0, _UCH // 2, pair, 0)

        plsc.subcore_barrier()
        pltpu.sync_copy(acc_sh.at[pl.ds(sid * rows, rows)],
                        acc_out.at[cid, pl.ds(sid * rows, rows)])

    return k(h, e, src2, dst2, c64, t16, zpad)


def _sc_ecat_call(h, src2, dst2):
    @functools.partial(
        pl.kernel,
        mesh=_sc_mesh(),
        out_type=jax.ShapeDtypeStruct((_IPAD * _CH, 2 * _H), jnp.float32),
        scratch_types=[
            pltpu.VMEM((2, 1, _CH), jnp.int32),
            pltpu.VMEM((2, 1, _CH), jnp.int32),
            pltpu.VMEM((_CH, 2 * _H), jnp.float32),
            pltpu.VMEM((_CH, 2 * _H), jnp.float32),
            pltpu.VMEM((_CH, 2 * _H), jnp.float32),
            pltpu.VMEM((_CH, 2 * _H), jnp.float32),
            pltpu.SemaphoreType.DMA,
            pltpu.SemaphoreType.DMA,
            pltpu.SemaphoreType.DMA,
            pltpu.SemaphoreType.DMA,
        ],
    )
    def k(h_hbm, src_hbm, dst_hbm, gcat_out,
          srcp, dstp, srow0, srow1, drow0, drow1,
          ssem0, ssem1, dsem0, dsem1):
        cid = lax.axis_index("c")
        sid = lax.axis_index("s")
        wid = sid * _NC + cid
        start = _UCH * wid

        def move(sr, dr):
            def body(i, carry):
                for q in range(4):
                    sr[i, pl.ds(_H + q * 16, 16)] = dr[i, pl.ds(q * 16, 16)]
                return carry
            lax.fori_loop(0, _CH, body, 0)

        def pair(p, carry):
            j0 = 2 * p
            g0 = start + j0
            pltpu.sync_copy(src_hbm.at[pl.ds(g0, 2)], srcp)
            pltpu.sync_copy(dst_hbm.at[pl.ds(g0, 2)], dstp)
            cp0 = pltpu.async_copy(h_hbm.at[srcp.at[0, 0]], srow0, ssem0)
            cp1 = pltpu.async_copy(h_hbm.at[dstp.at[0, 0]], drow0, dsem0)
            cp2 = pltpu.async_copy(h_hbm.at[srcp.at[1, 0]], srow1, ssem1)
            cp3 = pltpu.async_copy(h_hbm.at[dstp.at[1, 0]], drow1, dsem1)
            cp0.wait()
            cp1.wait()
            move(srow0, drow0)
            pltpu.sync_copy(srow0, gcat_out.at[pl.ds(g0 * _CH, _CH)])
            cp2.wait()
            cp3.wait()
            move(srow1, drow1)
            pltpu.sync_copy(srow1, gcat_out.at[pl.ds((g0 + 1) * _CH, _CH)])
            return carry
        lax.fori_loop(0, _UCH // 2, pair, 0)

    return k(h, src2, dst2)


def kernel(x, edge_index, edge_attr, enc_n_W0, enc_n_b0, enc_n_W1, enc_n_b1,
           enc_e_W0, enc_e_b0, enc_e_W1, enc_e_b1, t, mlp_W0, mlp_b0, mlp_g,
           mlp_bn, mlp_W1, mlp_b1, ln_g, ln_b, dec_n_W0, dec_n_b0, dec_n_W1,
           dec_n_b1, dec_e_W0, dec_e_b0, dec_e_W1, dec_e_b1):
    npadc = _IPAD - _NCHUNK
    src2 = jnp.concatenate(
        [edge_index[0].reshape(_NCHUNK, 1, _CH),
         jnp.zeros((npadc, 1, _CH), jnp.int32)])
    dst2 = jnp.concatenate(
        [edge_index[1].reshape(_NCHUNK, 1, _CH),
         jnp.full((npadc, 1, _CH), _DUMP, jnp.int32)])
    zpad = jnp.zeros((_CH, 2 * _H), jnp.float32)

    h, hmax = _mlp_enc_call(x, enc_n_W0, enc_n_b0, enc_n_W1, enc_n_b1,
                            bm=1000, pad_out=True)
    e, emax = _mlp_enc_call(edge_attr, enc_e_W0, enc_e_b0, enc_e_W1,
                            enc_e_b1, bm=2000, pad_out=False)

    ln_g_next = jnp.roll(ln_g, -1, axis=0)
    ln_b_next = jnp.roll(ln_b, -1, axis=0)

    def layer_step(carry, params):
        hprev, hin, cmax = carry
        (ti, W0, b0, g, bn, W1, b1, g2, b2) = params
        c = jnp.maximum(ti * (jnp.maximum(cmax + emax[0], 0.0) + 1e-7), 0.0)
        t16 = jnp.full((16,), ti, jnp.float32)
        acc = _sc_agg_call(hin, e, src2, dst2, c, t16, zpad)
        h_new, r_next, rmax = _layer_post_call(
            acc[0], acc[1], hin, hprev, W0, b0, g, bn, W1, b1, g2, b2)
        return (h_new, r_next, rmax[0]), None

    carry0 = (jnp.zeros((_N, 2 * _H), jnp.float32), h, hmax[0])
    params = (t, mlp_W0, mlp_b0, mlp_g, mlp_bn, mlp_W1, mlp_b1,
              ln_g_next, ln_b_next)
    (h_last, hfin, _), _ = lax.scan(layer_step, carry0, params)
    x_out = _dec_node_call(hfin, dec_n_W0, dec_n_b0, dec_n_W1, dec_n_b1)

    Gcat = _sc_ecat_call(hfin, src2, dst2)
    e_out = _dec_edge_call(Gcat, dec_e_W0, dec_e_b0, dec_e_W1, dec_e_b1)
    return (x_out, e_out)


# speedup vs baseline: 5.7542x; 1.0872x over previous
import functools

import jax
import jax.numpy as jnp
from jax import lax
from jax.experimental import pallas as pl
from jax.experimental.pallas import tpu as pltpu
from jax.experimental.pallas import tpu_sc as plsc

_N = 10000
_E = 320000
_H = 64
_NC = 2
_NS = 16
_NW = _NC * _NS
_CH = 80
_NCHUNK = _E // _CH
_UCH = 126
_IPAD = _NW * _UCH
_DUMP = 10239
_NPAD = 10240


def _mlp_enc_call(x, W0, b0, W1, b1, bm, pad_out):
    M, K = x.shape
    K2 = W1.shape[0]
    Dout = W1.shape[1]
    Wout = 2 * Dout if pad_out else Dout
    grid = (M // bm,)

    def kfn(x_ref, w0_ref, b0_ref, w1_ref, b1_ref, o_ref, mx_ref):
        z = jnp.dot(x_ref[...], w0_ref[...], preferred_element_type=jnp.float32)
        z = jnp.maximum(z + b0_ref[...], 0.0)
        h = jnp.dot(z, w1_ref[...], preferred_element_type=jnp.float32) + b1_ref[...]
        if pad_out:
            o_ref[...] = jnp.concatenate([h, jnp.zeros_like(h)], axis=-1)
        else:
            o_ref[...] = h
        cur = jnp.max(h, axis=0, keepdims=True)

        @pl.when(pl.program_id(0) == 0)
        def _():
            mx_ref[...] = cur

        @pl.when(pl.program_id(0) != 0)
        def _():
            mx_ref[...] = jnp.maximum(mx_ref[...], cur)

    return pl.pallas_call(
        kfn,
        grid=grid,
        in_specs=[
            pl.BlockSpec((bm, K), lambda i: (i, 0)),
            pl.BlockSpec((K, K2), lambda i: (0, 0)),
            pl.BlockSpec((1, K2), lambda i: (0, 0)),
            pl.BlockSpec((K2, Dout), lambda i: (0, 0)),
            pl.BlockSpec((1, Dout), lambda i: (0, 0)),
        ],
        out_specs=[
            pl.BlockSpec((bm, Wout), lambda i: (i, 0)),
            pl.BlockSpec((1, Dout), lambda i: (0, 0)),
        ],
        out_shape=[
            jax.ShapeDtypeStruct((M, Wout), jnp.float32),
            jax.ShapeDtypeStruct((1, Dout), jnp.float32),
        ],
    )(x, W0, b0.reshape(1, -1), W1, b1.reshape(1, -1))


def _ln(z, g, b):
    mu = jnp.mean(z, axis=-1, keepdims=True)
    var = jnp.mean((z - mu) ** 2, axis=-1, keepdims=True)
    return (z - mu) * jax.lax.rsqrt(var + 1e-5) * g + b


def _layer_post_call(acc0, acc1, hin, hprev, W0, b0, g, bn, W1, b1, g2, b2):
    bm = 1000
    grid = (_N // bm,)

    def kfn(a0, a1, hi, hp, w0, b0r, gr, bnr, w1, b1r, g2r, b2r,
            o0, o1, o2):
        av0 = a0[...]
        av1 = a1[...]
        num = av0[:, :_H] + av1[:, :_H]
        den = av0[:, _H:] + av1[:, _H:]
        ok = den > 0.0
        agg = jnp.where(ok, num / jnp.where(ok, den, 1.0), 0.0)
        out = agg + hi[...][:, :_H]
        z = jnp.dot(out, w0[...], preferred_element_type=jnp.float32) + b0r[...]
        z = jnp.maximum(_ln(z, gr[...], bnr[...]), 0.0)
        conv = jnp.dot(z, w1[...], preferred_element_type=jnp.float32) + b1r[...]
        h_new = hp[...][:, :_H] + conv
        post = jnp.maximum(_ln(h_new, g2r[...], b2r[...]), 0.0)
        zpad = jnp.zeros_like(post)
        o0[...] = jnp.concatenate([h_new, zpad], axis=-1)
        o1[...] = jnp.concatenate([post, zpad], axis=-1)
        cur = jnp.max(post, axis=0, keepdims=True)

        @pl.when(pl.program_id(0) == 0)
        def _():
            o2[...] = cur

        @pl.when(pl.program_id(0) != 0)
        def _():
            o2[...] = jnp.maximum(o2[...], cur)

    wide_spec = pl.BlockSpec((bm, 2 * _H), lambda i: (i, 0))
    full = lambda r, c: pl.BlockSpec((r, c), lambda i: (0, 0))
    in_specs = [wide_spec, wide_spec, wide_spec, wide_spec,
                full(_H, 2 * _H), full(1, 2 * _H), full(1, 2 * _H),
                full(1, 2 * _H), full(2 * _H, _H), full(1, _H),
                full(1, _H), full(1, _H)]
    args = [acc0, acc1, hin, hprev,
            W0, b0.reshape(1, -1), g.reshape(1, -1), bn.reshape(1, -1),
            W1, b1.reshape(1, -1), g2.reshape(1, -1), b2.reshape(1, -1)]
    out_specs = [wide_spec, wide_spec, pl.BlockSpec((1, _H), lambda i: (0, 0))]
    out_shape = [jax.ShapeDtypeStruct((_N, 2 * _H), jnp.float32),
                 jax.ShapeDtypeStruct((_N, 2 * _H), jnp.float32),
                 jax.ShapeDtypeStruct((1, _H), jnp.float32)]

    return pl.pallas_call(
        kfn, grid=grid, in_specs=in_specs, out_specs=out_specs,
        out_shape=out_shape,
    )(*args)


def _dec_node_call(hfin, dW0, db0, dW1, db1):
    bm = 1000
    grid = (_N // bm,)

    def kfn(hi, dw0, db0r, dw1, db1r, o_ref):
        hv = hi[...][:, :_H]
        zz = jnp.dot(hv, dw0[...], preferred_element_type=jnp.float32)
        zz = jnp.maximum(zz + db0r[...], 0.0)
        o_ref[...] = jnp.dot(zz, dw1[...], preferred_element_type=jnp.float32) + db1r[...]

    full = lambda r, c: pl.BlockSpec((r, c), lambda i: (0, 0))
    return pl.pallas_call(
        kfn, grid=grid,
        in_specs=[pl.BlockSpec((bm, 2 * _H), lambda i: (i, 0)),
                  full(_H, 128), full(1, 128), full(128, 2), full(1, 2)],
        out_specs=pl.BlockSpec((bm, 2), lambda i: (i, 0)),
        out_shape=jax.ShapeDtypeStruct((_N, 2), jnp.float32),
    )(hfin, dW0, db0.reshape(1, -1), dW1, db1.reshape(1, -1))


def _dec_edge_call(Gcat, W0, b0, W1, b1):
    be = 2000
    grid = (_E // be,)

    def kfn(gc, w0, b0r, w1, b1r, o_ref):
        z = jnp.dot(gc[...], w0[...], preferred_element_type=jnp.float32)
        z = jnp.maximum(z + b0r[...], 0.0)
        o_ref[...] = jnp.dot(z, w1[...], preferred_element_type=jnp.float32) + b1r[...]

    full = lambda r, c: pl.BlockSpec((r, c), lambda i: (0, 0))
    return pl.pallas_call(
        kfn, grid=grid,
        in_specs=[
            pl.BlockSpec((be, 2 * _H), lambda i: (i, 0)),
            full(2 * _H, 256), full(1, 256), full(256, 2), full(1, 2),
        ],
        out_specs=pl.BlockSpec((be, 2), lambda i: (i, 0)),
        out_shape=jax.ShapeDtypeStruct((_E, 2), jnp.float32),
    )(Gcat, W0, b0.reshape(1, -1), W1, b1.reshape(1, -1))


def _sc_mesh():
    return plsc.VectorSubcoreMesh(core_axis_name="c", subcore_axis_name="s")


def _sc_agg_call(h, e, sd2, c64, t16, zpad):
    @functools.partial(
        pl.kernel,
        mesh=_sc_mesh(),
        out_type=jax.ShapeDtypeStruct((_NC, _NPAD, 2 * _H), jnp.float32),
        scratch_types=[
            pltpu.VMEM((2, 2, _CH), jnp.int32),
            pltpu.VMEM((_CH, 2 * _H), jnp.float32),
            pltpu.VMEM((_CH, 2 * _H), jnp.float32),
            pltpu.VMEM((_CH, _H), jnp.float32),
            pltpu.VMEM((_CH, _H), jnp.float32),
            pltpu.VMEM((_H,), jnp.float32),
            pltpu.VMEM((16,), jnp.float32),
            pltpu.VMEM_SHARED((_NPAD, 2 * _H), jnp.float32),
            pltpu.SemaphoreType.DMA,
            pltpu.SemaphoreType.DMA,
            pltpu.SemaphoreType.DMA,
            pltpu.SemaphoreType.DMA,
            pltpu.SemaphoreType.DMA,
        ],
    )
    def k(h_hbm, e_hbm, sd_hbm, c_hbm, t_hbm, z_hbm, acc_out,
          sdp, hrow0, hrow1, erow0, erow1,
          c_v, t_v, acc_sh, gsem0, gsem1, esem0, esem1, ssem):
        cid = lax.axis_index("c")
        sid = lax.axis_index("s")
        wid = sid * _NC + cid
        rows = _NPAD // _NS
        for zi in range(rows // _CH):
            pltpu.sync_copy(
                z_hbm, acc_sh.at[pl.ds(sid * rows + zi * _CH, _CH)])
        pltpu.sync_copy(c_hbm, c_v)
        pltpu.sync_copy(t_hbm, t_v)
        start = _UCH * wid
        plsc.subcore_barrier()

        def compute(hr, er):
            def body(i, carry):
                c0, c1, c2, c3, tv = carry
                cs = (c0, c1, c2, c3)
                for q in range(4):
                    hv = hr[i, pl.ds(q * 16, 16)]
                    ev = er[i, pl.ds(q * 16, 16)]
                    m = jnp.maximum(hv + ev, 0.0) + 1e-7
                    em = jnp.exp(m * tv - cs[q])
                    hr[i, pl.ds(q * 16, 16)] = m * em
                    hr[i, pl.ds(_H + q * 16, 16)] = em
                return carry
            carry0 = (c_v[pl.ds(0, 16)], c_v[pl.ds(16, 16)],
                      c_v[pl.ds(32, 16)], c_v[pl.ds(48, 16)], t_v[...])
            lax.fori_loop(0, _CH, body, carry0)

        def pair(p, carry):
            j0 = 2 * p
            g0 = start + j0
            g1 = g0 + 1
            ge0 = jnp.where(g0 < _NCHUNK, g0, 0)
            ge1 = jnp.where(g1 < _NCHUNK, g1, 0)
            pltpu.sync_copy(sd_hbm.at[pl.ds(g0, 2)], sdp)
            cp0 = pltpu.async_copy(h_hbm.at[sdp.at[0, 0]], hrow0, gsem0)
            cp1 = pltpu.async_copy(e_hbm.at[pl.ds(ge0 * _CH, _CH)], erow0,
                                   esem0)
            cp2 = pltpu.async_copy(h_hbm.at[sdp.at[1, 0]], hrow1, gsem1)
            cp3 = pltpu.async_copy(e_hbm.at[pl.ds(ge1 * _CH, _CH)], erow1,
                                   esem1)
            cp0.wait()
            cp1.wait()
            compute(hrow0, erow0)
            sc0 = pltpu.async_copy(hrow0, acc_sh.at[sdp.at[0, 1]], ssem,
                                   add=True)
            cp2.wait()
            cp3.wait()
            compute(hrow1, erow1)
            sc0.wait()
            pltpu.sync_copy(hrow1, acc_sh.at[sdp.at[1, 1]], add=True)
            return carry
        lax.fori_loop(0, _UCH // 2, pair, 0)

        plsc.subcore_barrier()
        pltpu.sync_copy(acc_sh.at[pl.ds(sid * rows, rows)],
                        acc_out.at[cid, pl.ds(sid * rows, rows)])

    return k(h, e, sd2, c64, t16, zpad)


def _sc_ecat_call(h, sd2):
    @functools.partial(
        pl.kernel,
        mesh=_sc_mesh(),
        out_type=jax.ShapeDtypeStruct((_IPAD * _CH, 2 * _H), jnp.float32),
        scratch_types=[
            pltpu.VMEM((2, 2, _CH), jnp.int32),
            pltpu.VMEM((_CH, 2 * _H), jnp.float32),
            pltpu.VMEM((_CH, 2 * _H), jnp.float32),
            pltpu.VMEM((_CH, 2 * _H), jnp.float32),
            pltpu.VMEM((_CH, 2 * _H), jnp.float32),
            pltpu.SemaphoreType.DMA,
            pltpu.SemaphoreType.DMA,
            pltpu.SemaphoreType.DMA,
            pltpu.SemaphoreType.DMA,
            pltpu.SemaphoreType.DMA,
        ],
    )
    def k(h_hbm, sd_hbm, gcat_out,
          sdp, srow0, srow1, drow0, drow1,
          ssem0, ssem1, dsem0, dsem1, wsem):
        cid = lax.axis_index("c")
        sid = lax.axis_index("s")
        wid = sid * _NC + cid
        start = _UCH * wid

        def move(sr, dr):
            def body(i, carry):
                for q in range(4):
                    sr[i, pl.ds(_H + q * 16, 16)] = dr[i, pl.ds(q * 16, 16)]
                return carry
            lax.fori_loop(0, _CH, body, 0)

        def pair(p, carry):
            j0 = 2 * p
            g0 = start + j0
            pltpu.sync_copy(sd_hbm.at[pl.ds(g0, 2)], sdp)
            cp0 = pltpu.async_copy(h_hbm.at[sdp.at[0, 0]], srow0, ssem0)
            cp1 = pltpu.async_copy(h_hbm.at[sdp.at[0, 1]], drow0, dsem0)
            cp2 = pltpu.async_copy(h_hbm.at[sdp.at[1, 0]], srow1, ssem1)
            cp3 = pltpu.async_copy(h_hbm.at[sdp.at[1, 1]], drow1, dsem1)
            cp0.wait()
            cp1.wait()
            move(srow0, drow0)
            w0 = pltpu.async_copy(srow0, gcat_out.at[pl.ds(g0 * _CH, _CH)],
                                  wsem)
            cp2.wait()
            cp3.wait()
            move(srow1, drow1)
            w0.wait()
            pltpu.sync_copy(srow1, gcat_out.at[pl.ds((g0 + 1) * _CH, _CH)])
            return carry
        lax.fori_loop(0, _UCH // 2, pair, 0)

    return k(h, sd2)


def kernel(x, edge_index, edge_attr, enc_n_W0, enc_n_b0, enc_n_W1, enc_n_b1,
           enc_e_W0, enc_e_b0, enc_e_W1, enc_e_b1, t, mlp_W0, mlp_b0, mlp_g,
           mlp_bn, mlp_W1, mlp_b1, ln_g, ln_b, dec_n_W0, dec_n_b0, dec_n_W1,
           dec_n_b1, dec_e_W0, dec_e_b0, dec_e_W1, dec_e_b1):
    npadc = _IPAD - _NCHUNK
    src2 = jnp.concatenate(
        [edge_index[0].reshape(_NCHUNK, 1, _CH),
         jnp.zeros((npadc, 1, _CH), jnp.int32)])
    dst2 = jnp.concatenate(
        [edge_index[1].reshape(_NCHUNK, 1, _CH),
         jnp.full((npadc, 1, _CH), _DUMP, jnp.int32)])
    sd2 = jnp.concatenate([src2, dst2], axis=1)
    zpad = jnp.zeros((_CH, 2 * _H), jnp.float32)

    h, hmax = _mlp_enc_call(x, enc_n_W0, enc_n_b0, enc_n_W1, enc_n_b1,
                            bm=1000, pad_out=True)
    e, emax = _mlp_enc_call(edge_attr, enc_e_W0, enc_e_b0, enc_e_W1,
                            enc_e_b1, bm=2000, pad_out=False)

    ln_g_next = jnp.roll(ln_g, -1, axis=0)
    ln_b_next = jnp.roll(ln_b, -1, axis=0)

    def layer_step(carry, params):
        hprev, hin, cmax = carry
        (ti, W0, b0, g, bn, W1, b1, g2, b2) = params
        c = jnp.maximum(ti * (jnp.maximum(cmax + emax[0], 0.0) + 1e-7), 0.0)
        t16 = jnp.full((16,), ti, jnp.float32)
        acc = _sc_agg_call(hin, e, sd2, c, t16, zpad)
        h_new, r_next, rmax = _layer_post_call(
            acc[0], acc[1], hin, hprev, W0, b0, g, bn, W1, b1, g2, b2)
        return (h_new, r_next, rmax[0]), None

    carry0 = (jnp.zeros((_N, 2 * _H), jnp.float32), h, hmax[0])
    params = (t, mlp_W0, mlp_b0, mlp_g, mlp_bn, mlp_W1, mlp_b1,
              ln_g_next, ln_b_next)
    (h_last, hfin, _), _ = lax.scan(layer_step, carry0, params)
    x_out = _dec_node_call(hfin, dec_n_W0, dec_n_b0, dec_n_W1, dec_n_b1)

    Gcat = _sc_ecat_call(hfin, sd2)
    e_out = _dec_edge_call(Gcat, dec_e_W0, dec_e_b0, dec_e_W1, dec_e_b1)
    return (x_out, e_out)


# speedup vs baseline: 6.4352x; 1.1183x over previous
import functools

import jax
import jax.numpy as jnp
from jax import lax
from jax.experimental import pallas as pl
from jax.experimental.pallas import tpu as pltpu
from jax.experimental.pallas import tpu_sc as plsc

_N = 10000
_E = 320000
_H = 64
_NC = 2
_NS = 16
_NW = _NC * _NS
_CH = 80
_NCHUNK = _E // _CH
_UCH = 126
_IPAD = _NW * _UCH
_DUMP = 10239
_NPAD = 10240


def _mlp_enc_call(x, W0, b0, W1, b1, bm, pad_out):
    M, K = x.shape
    K2 = W1.shape[0]
    Dout = W1.shape[1]
    Wout = 2 * Dout if pad_out else Dout
    grid = (M // bm,)

    def kfn(x_ref, w0_ref, b0_ref, w1_ref, b1_ref, o_ref, mx_ref):
        z = jnp.dot(x_ref[...], w0_ref[...], preferred_element_type=jnp.float32)
        z = jnp.maximum(z + b0_ref[...], 0.0)
        h = jnp.dot(z, w1_ref[...], preferred_element_type=jnp.float32) + b1_ref[...]
        if pad_out:
            o_ref[...] = jnp.concatenate([h, jnp.zeros_like(h)], axis=-1)
        else:
            o_ref[...] = h
        cur = jnp.max(h, axis=0, keepdims=True)

        @pl.when(pl.program_id(0) == 0)
        def _():
            mx_ref[...] = cur

        @pl.when(pl.program_id(0) != 0)
        def _():
            mx_ref[...] = jnp.maximum(mx_ref[...], cur)

    return pl.pallas_call(
        kfn,
        grid=grid,
        in_specs=[
            pl.BlockSpec((bm, K), lambda i: (i, 0)),
            pl.BlockSpec((K, K2), lambda i: (0, 0)),
            pl.BlockSpec((1, K2), lambda i: (0, 0)),
            pl.BlockSpec((K2, Dout), lambda i: (0, 0)),
            pl.BlockSpec((1, Dout), lambda i: (0, 0)),
        ],
        out_specs=[
            pl.BlockSpec((bm, Wout), lambda i: (i, 0)),
            pl.BlockSpec((1, Dout), lambda i: (0, 0)),
        ],
        out_shape=[
            jax.ShapeDtypeStruct((M, Wout), jnp.float32),
            jax.ShapeDtypeStruct((1, Dout), jnp.float32),
        ],
    )(x, W0, b0.reshape(1, -1), W1, b1.reshape(1, -1))


def _ln(z, g, b):
    mu = jnp.mean(z, axis=-1, keepdims=True)
    var = jnp.mean((z - mu) ** 2, axis=-1, keepdims=True)
    return (z - mu) * jax.lax.rsqrt(var + 1e-5) * g + b


def _layer_post_call(acc0, acc1, hin, hprev, W0, b0, g, bn, W1, b1, g2, b2):
    bm = 1000
    grid = (_N // bm,)

    def kfn(a0, a1, hi, hp, w0, b0r, gr, bnr, w1, b1r, g2r, b2r,
            o0, o1, o2):
        av0 = a0[...]
        av1 = a1[...]
        num = av0[:, :_H] + av1[:, :_H]
        den = av0[:, _H:] + av1[:, _H:]
        ok = den > 0.0
        agg = jnp.where(ok, num / jnp.where(ok, den, 1.0), 0.0)
        out = agg + hi[...][:, :_H]
        z = jnp.dot(out, w0[...], preferred_element_type=jnp.float32) + b0r[...]
        z = jnp.maximum(_ln(z, gr[...], bnr[...]), 0.0)
        conv = jnp.dot(z, w1[...], preferred_element_type=jnp.float32) + b1r[...]
        h_new = hp[...][:, :_H] + conv
        post = jnp.maximum(_ln(h_new, g2r[...], b2r[...]), 0.0)
        zpad = jnp.zeros_like(post)
        o0[...] = jnp.concatenate([h_new, zpad], axis=-1)
        o1[...] = jnp.concatenate([post, zpad], axis=-1)
        cur = jnp.max(post, axis=0, keepdims=True)

        @pl.when(pl.program_id(0) == 0)
        def _():
            o2[...] = cur

        @pl.when(pl.program_id(0) != 0)
        def _():
            o2[...] = jnp.maximum(o2[...], cur)

    wide_spec = pl.BlockSpec((bm, 2 * _H), lambda i: (i, 0))
    full = lambda r, c: pl.BlockSpec((r, c), lambda i: (0, 0))
    in_specs = [wide_spec, wide_spec, wide_spec, wide_spec,
                full(_H, 2 * _H), full(1, 2 * _H), full(1, 2 * _H),
                full(1, 2 * _H), full(2 * _H, _H), full(1, _H),
                full(1, _H), full(1, _H)]
    args = [acc0, acc1, hin, hprev,
            W0, b0.reshape(1, -1), g.reshape(1, -1), bn.reshape(1, -1),
            W1, b1.reshape(1, -1), g2.reshape(1, -1), b2.reshape(1, -1)]
    out_specs = [wide_spec, wide_spec, pl.BlockSpec((1, _H), lambda i: (0, 0))]
    out_shape = [jax.ShapeDtypeStruct((_N, 2 * _H), jnp.float32),
                 jax.ShapeDtypeStruct((_N, 2 * _H), jnp.float32),
                 jax.ShapeDtypeStruct((1, _H), jnp.float32)]

    return pl.pallas_call(
        kfn, grid=grid, in_specs=in_specs, out_specs=out_specs,
        out_shape=out_shape,
    )(*args)


def _dec_node_call(hfin, dW0, db0, dW1, db1):
    bm = 1000
    grid = (_N // bm,)

    def kfn(hi, dw0, db0r, dw1, db1r, o_ref):
        hv = hi[...][:, :_H]
        zz = jnp.dot(hv, dw0[...], preferred_element_type=jnp.float32)
        zz = jnp.maximum(zz + db0r[...], 0.0)
        o_ref[...] = jnp.dot(zz, dw1[...], preferred_element_type=jnp.float32) + db1r[...]

    full = lambda r, c: pl.BlockSpec((r, c), lambda i: (0, 0))
    return pl.pallas_call(
        kfn, grid=grid,
        in_specs=[pl.BlockSpec((bm, 2 * _H), lambda i: (i, 0)),
                  full(_H, 128), full(1, 128), full(128, 2), full(1, 2)],
        out_specs=pl.BlockSpec((bm, 2), lambda i: (i, 0)),
        out_shape=jax.ShapeDtypeStruct((_N, 2), jnp.float32),
    )(hfin, dW0, db0.reshape(1, -1), dW1, db1.reshape(1, -1))


def _dec_edge_call(Gcat, W0, b0, W1, b1):
    be = 2000
    grid = (_E // be,)

    def kfn(gc, w0, b0r, w1, b1r, o_ref):
        z = jnp.dot(gc[...], w0[...], preferred_element_type=jnp.float32)
        z = jnp.maximum(z + b0r[...], 0.0)
        o_ref[...] = jnp.dot(z, w1[...], preferred_element_type=jnp.float32) + b1r[...]

    full = lambda r, c: pl.BlockSpec((r, c), lambda i: (0, 0))
    return pl.pallas_call(
        kfn, grid=grid,
        in_specs=[
            pl.BlockSpec((be, 2 * _H), lambda i: (i, 0)),
            full(2 * _H, 256), full(1, 256), full(256, 2), full(1, 2),
        ],
        out_specs=pl.BlockSpec((be, 2), lambda i: (i, 0)),
        out_shape=jax.ShapeDtypeStruct((_E, 2), jnp.float32),
    )(Gcat, W0, b0.reshape(1, -1), W1, b1.reshape(1, -1))


def _sc_mesh():
    return plsc.VectorSubcoreMesh(core_axis_name="c", subcore_axis_name="s")


def _sc_agg_call(h, e, sd2, c64, t16, zpad):
    @functools.partial(
        pl.kernel,
        mesh=_sc_mesh(),
        out_type=jax.ShapeDtypeStruct((_NC, _NPAD, 2 * _H), jnp.float32),
        scratch_types=[
            pltpu.VMEM((2, 2, _CH), jnp.int32),
            pltpu.VMEM((_CH, 2 * _H), jnp.float32),
            pltpu.VMEM((_CH, 2 * _H), jnp.float32),
            pltpu.VMEM((_CH, _H), jnp.float32),
            pltpu.VMEM((_CH, _H), jnp.float32),
            pltpu.VMEM((_H,), jnp.float32),
            pltpu.VMEM((16,), jnp.float32),
            pltpu.VMEM_SHARED((_NPAD, 2 * _H), jnp.float32),
            pltpu.SemaphoreType.DMA,
            pltpu.SemaphoreType.DMA,
            pltpu.SemaphoreType.DMA,
            pltpu.SemaphoreType.DMA,
            pltpu.SemaphoreType.DMA,
        ],
    )
    def k(h_hbm, e_hbm, sd_hbm, c_hbm, t_hbm, z_hbm, acc_out,
          sdp, hrow0, hrow1, erow0, erow1,
          c_v, t_v, acc_sh, gsem0, gsem1, esem0, esem1, ssem):
        cid = lax.axis_index("c")
        sid = lax.axis_index("s")
        wid = sid * _NC + cid
        rows = _NPAD // _NS
        for zi in range(rows // _CH):
            pltpu.sync_copy(
                z_hbm, acc_sh.at[pl.ds(sid * rows + zi * _CH, _CH)])
        pltpu.sync_copy(c_hbm, c_v)
        pltpu.sync_copy(t_hbm, t_v)
        start = _UCH * wid
        plsc.subcore_barrier()

        def compute(hr, er):
            def body(i8, carry):
                c0, c1, c2, c3, tv = carry
                cs = (c0, c1, c2, c3)
                for r in range(8):
                    i = i8 * 8 + r
                    for q in range(4):
                        hv = hr[i, pl.ds(q * 16, 16)]
                        ev = er[i, pl.ds(q * 16, 16)]
                        m = jnp.maximum(hv + ev, 0.0) + 1e-7
                        em = jnp.exp(m * tv - cs[q])
                        hr[i, pl.ds(q * 16, 16)] = m * em
                        hr[i, pl.ds(_H + q * 16, 16)] = em
                return carry
            carry0 = (c_v[pl.ds(0, 16)], c_v[pl.ds(16, 16)],
                      c_v[pl.ds(32, 16)], c_v[pl.ds(48, 16)], t_v[...])
            lax.fori_loop(0, _CH // 8, body, carry0)

        def pair(p, carry):
            j0 = 2 * p
            g0 = start + j0
            g1 = g0 + 1
            ge0 = jnp.where(g0 < _NCHUNK, g0, 0)
            ge1 = jnp.where(g1 < _NCHUNK, g1, 0)
            pltpu.sync_copy(sd_hbm.at[pl.ds(g0, 2)], sdp)
            cp0 = pltpu.async_copy(h_hbm.at[sdp.at[0, 0]], hrow0, gsem0)
            cp1 = pltpu.async_copy(e_hbm.at[pl.ds(ge0 * _CH, _CH)], erow0,
                                   esem0)
            cp2 = pltpu.async_copy(h_hbm.at[sdp.at[1, 0]], hrow1, gsem1)
            cp3 = pltpu.async_copy(e_hbm.at[pl.ds(ge1 * _CH, _CH)], erow1,
                                   esem1)
            cp0.wait()
            cp1.wait()
            compute(hrow0, erow0)
            sc0 = pltpu.async_copy(hrow0, acc_sh.at[sdp.at[0, 1]], ssem,
                                   add=True)
            cp2.wait()
            cp3.wait()
            compute(hrow1, erow1)
            sc0.wait()
            pltpu.sync_copy(hrow1, acc_sh.at[sdp.at[1, 1]], add=True)
            return carry
        lax.fori_loop(0, _UCH // 2, pair, 0)

        plsc.subcore_barrier()
        pltpu.sync_copy(acc_sh.at[pl.ds(sid * rows, rows)],
                        acc_out.at[cid, pl.ds(sid * rows, rows)])

    return k(h, e, sd2, c64, t16, zpad)


def _sc_ecat_call(h, sd2):
    @functools.partial(
        pl.kernel,
        mesh=_sc_mesh(),
        out_type=jax.ShapeDtypeStruct((_IPAD * _CH, 2 * _H), jnp.float32),
        scratch_types=[
            pltpu.VMEM((2, 2, _CH), jnp.int32),
            pltpu.VMEM((_CH, 2 * _H), jnp.float32),
            pltpu.VMEM((_CH, 2 * _H), jnp.float32),
            pltpu.VMEM((_CH, 2 * _H), jnp.float32),
            pltpu.VMEM((_CH, 2 * _H), jnp.float32),
            pltpu.SemaphoreType.DMA,
            pltpu.SemaphoreType.DMA,
            pltpu.SemaphoreType.DMA,
            pltpu.SemaphoreType.DMA,
            pltpu.SemaphoreType.DMA,
        ],
    )
    def k(h_hbm, sd_hbm, gcat_out,
          sdp, srow0, srow1, drow0, drow1,
          ssem0, ssem1, dsem0, dsem1, wsem):
        cid = lax.axis_index("c")
        sid = lax.axis_index("s")
        wid = sid * _NC + cid
        start = _UCH * wid

        def move(sr, dr):
            def body(i8, carry):
                for r in range(8):
                    i = i8 * 8 + r
                    for q in range(4):
                        sr[i, pl.ds(_H + q * 16, 16)] = dr[i, pl.ds(q * 16, 16)]
                return carry
            lax.fori_loop(0, _CH // 8, body, 0)

        def pair(p, carry):
            j0 = 2 * p
            g0 = start + j0
            pltpu.sync_copy(sd_hbm.at[pl.ds(g0, 2)], sdp)
            cp0 = pltpu.async_copy(h_hbm.at[sdp.at[0, 0]], srow0, ssem0)
            cp1 = pltpu.async_copy(h_hbm.at[sdp.at[0, 1]], drow0, dsem0)
            cp2 = pltpu.async_copy(h_hbm.at[sdp.at[1, 0]], srow1, ssem1)
            cp3 = pltpu.async_copy(h_hbm.at[sdp.at[1, 1]], drow1, dsem1)
            cp0.wait()
            cp1.wait()
            move(srow0, drow0)
            w0 = pltpu.async_copy(srow0, gcat_out.at[pl.ds(g0 * _CH, _CH)],
                                  wsem)
            cp2.wait()
            cp3.wait()
            move(srow1, drow1)
            w0.wait()
            pltpu.sync_copy(srow1, gcat_out.at[pl.ds((g0 + 1) * _CH, _CH)])
            return carry
        lax.fori_loop(0, _UCH // 2, pair, 0)

    return k(h, sd2)


def kernel(x, edge_index, edge_attr, enc_n_W0, enc_n_b0, enc_n_W1, enc_n_b1,
           enc_e_W0, enc_e_b0, enc_e_W1, enc_e_b1, t, mlp_W0, mlp_b0, mlp_g,
           mlp_bn, mlp_W1, mlp_b1, ln_g, ln_b, dec_n_W0, dec_n_b0, dec_n_W1,
           dec_n_b1, dec_e_W0, dec_e_b0, dec_e_W1, dec_e_b1):
    npadc = _IPAD - _NCHUNK
    src2 = jnp.concatenate(
        [edge_index[0].reshape(_NCHUNK, 1, _CH),
         jnp.zeros((npadc, 1, _CH), jnp.int32)])
    dst2 = jnp.concatenate(
        [edge_index[1].reshape(_NCHUNK, 1, _CH),
         jnp.full((npadc, 1, _CH), _DUMP, jnp.int32)])
    sd2 = jnp.concatenate([src2, dst2], axis=1)
    zpad = jnp.zeros((_CH, 2 * _H), jnp.float32)

    h, hmax = _mlp_enc_call(x, enc_n_W0, enc_n_b0, enc_n_W1, enc_n_b1,
                            bm=1000, pad_out=True)
    e, emax = _mlp_enc_call(edge_attr, enc_e_W0, enc_e_b0, enc_e_W1,
                            enc_e_b1, bm=2000, pad_out=False)

    ln_g_next = jnp.roll(ln_g, -1, axis=0)
    ln_b_next = jnp.roll(ln_b, -1, axis=0)

    def layer_step(carry, params):
        hprev, hin, cmax = carry
        (ti, W0, b0, g, bn, W1, b1, g2, b2) = params
        c = jnp.maximum(ti * (jnp.maximum(cmax + emax[0], 0.0) + 1e-7), 0.0)
        t16 = jnp.full((16,), ti, jnp.float32)
        acc = _sc_agg_call(hin, e, sd2, c, t16, zpad)
        h_new, r_next, rmax = _layer_post_call(
            acc[0], acc[1], hin, hprev, W0, b0, g, bn, W1, b1, g2, b2)
        return (h_new, r_next, rmax[0]), None

    carry0 = (jnp.zeros((_N, 2 * _H), jnp.float32), h, hmax[0])
    params = (t, mlp_W0, mlp_b0, mlp_g, mlp_bn, mlp_W1, mlp_b1,
              ln_g_next, ln_b_next)
    (h_last, hfin, _), _ = lax.scan(layer_step, carry0, params)
    x_out = _dec_node_call(hfin, dec_n_W0, dec_n_b0, dec_n_W1, dec_n_b1)

    Gcat = _sc_ecat_call(hfin, sd2)
    e_out = _dec_edge_call(Gcat, dec_e_W0, dec_e_b0, dec_e_W1, dec_e_b1)
    return (x_out, e_out)
